# baseline (speedup 1.0000x reference)
import jax
import jax.numpy as jnp
from jax.experimental import pallas as pl
from jax.experimental.pallas import tpu as pltpu
from jax.experimental.pallas import tpu_sc as plsc

_B, _L, _D = 16384, 200, 128
_N = _B * _L
_HOURS = 72
_HSTRIDE = 64
_W = 128
_NROWS = _HOURS * _HSTRIDE
_RPS = _NROWS // 16


def _sc_gather(minute_table, hour_table, x0, x1):
    mesh = plsc.VectorSubcoreMesh(
        core_axis_name="core", subcore_axis_name="subcore"
    )

    @pl.kernel(
        out_type=jax.ShapeDtypeStruct((_N, _D), jnp.float32),
        mesh=mesh,
        scratch_types=[
            pltpu.VMEM((_W,), jnp.int32),
            pltpu.VMEM((_HSTRIDE, _D), jnp.float32),
            pltpu.VMEM((_HOURS, _D), jnp.float32),
            pltpu.VMEM((_RPS, _D), jnp.float32),
            pltpu.VMEM_SHARED((_NROWS, _D), jnp.float32),
        ],
    )
    def k(min_hbm, hour_hbm, x0_hbm, x1_hbm, out_hbm,
          idx_ref, min_v, hour_v, cbuf, combo_sh):
        sid = jax.lax.axis_index("subcore")
        pltpu.sync_copy(min_hbm, min_v.at[pl.ds(0, 60)])
        pltpu.sync_copy(hour_hbm, hour_v)
        base = sid * _RPS

        @pl.loop(0, _RPS)
        def _(r):
            row = base + r
            h = jax.lax.shift_right_logical(row, 6)
            m = jax.lax.bitwise_and(row, _HSTRIDE - 1)
            for i in range(_D // 16):
                s = pl.ds(i * 16, 16)
                cbuf[r, s] = hour_v[h, s] + min_v[m, s]

        sl = pl.ds(base, _RPS)
        pltpu.sync_copy(cbuf, combo_sh.at[sl])
        plsc.subcore_barrier()

        def body(x0_v, x1_v, o_v):
            x0r = x0_v.at[0]
            x1r = x1_v.at[0]
            for i in range(_W // 16):
                s = pl.ds(i * 16, 16)
                idx_ref[s] = x0r[s] * _HSTRIDE + x1r[s]
            pltpu.sync_copy(combo_sh.at[idx_ref], o_v)

        pltpu.emit_pipeline(
            body,
            grid=(_N // _W,),
            in_specs=[
                pl.BlockSpec((1, _W), lambda i: (0, i)),
                pl.BlockSpec((1, _W), lambda i: (0, i)),
            ],
            out_specs=[pl.BlockSpec((_W, _D), lambda i: (i, 0))],
            core_axis_name=("core", "subcore"),
            dimension_semantics=(pltpu.PARALLEL,),
        )(x0_hbm, x1_hbm, out_hbm)

    return k(minute_table, hour_table, x0, x1)


def kernel(x, minute_table, hour_table):
    x = x.astype(jnp.int32)
    x0 = x[:, :, 0].reshape(1, _N)
    x1 = x[:, :, 1].reshape(1, _N)
    out = _sc_gather(minute_table, hour_table, x0, x1)
    return out.reshape(_B, _L, _D)
